# baseline (speedup 1.0000x reference)
import functools

import jax
import jax.numpy as jnp
from jax import lax
from jax.experimental import pallas as pl
from jax.experimental.pallas import tpu as pltpu
from jax.experimental.pallas import tpu_sc as plsc

_NC = 2
_NS = 16
_NW = _NC * _NS

_CH = 128
_LANES = 16


def kernel(input_ids, table):
    B, H = input_ids.shape
    V, D = table.shape
    n_bhi = B // _CH
    n_cells = H * n_bhi
    per_w = n_cells // _NW
    fhi = D // 8

    ids2 = input_ids.T.reshape(n_cells, _CH)

    mesh = plsc.VectorSubcoreMesh(core_axis_name="c", subcore_axis_name="s")

    @functools.partial(
        pl.kernel,
        out_type=jax.ShapeDtypeStruct((H, fhi, n_bhi, 8, _CH), table.dtype),
        mesh=mesh,
        scratch_types=[
            pltpu.VMEM((per_w, _CH), jnp.int32),
            pltpu.VMEM((_CH, D), jnp.float32),
            pltpu.VMEM((_CH, D), jnp.float32),
            pltpu.VMEM((fhi, 8, _CH), jnp.float32),
            pltpu.VMEM((fhi, 8, _CH), jnp.float32),
            pltpu.SemaphoreType.DMA,
            pltpu.SemaphoreType.DMA,
            pltpu.SemaphoreType.DMA,
            pltpu.SemaphoreType.DMA,
        ],
        compiler_params=pltpu.CompilerParams(
            use_tc_tiling_on_sc=False, needs_layout_passes=False
        ),
    )
    def run(ids_hbm, table_hbm, out_hbm, idx_v, rows0, rows1, cell0, cell1,
            sg0, sg1, so0, so1):
        wid = lax.axis_index("s") * _NC + lax.axis_index("c")
        base = wid * per_w
        pltpu.sync_copy(ids_hbm.at[pl.ds(base, per_w)], idx_v)

        rows = (rows0, rows1)
        cells = (cell0, cell1)
        sgs = (sg0, sg1)
        sos = (so0, so1)

        iota = lax.iota(jnp.int32, _LANES)
        b_idx = [iota + (bc * _LANES) for bc in range(_CH // _LANES)]

        def fire_gather(l, b):
            pltpu.async_copy(table_hbm.at[idx_v.at[l]], rows[b], sgs[b])

        def wait_gather(l, b):
            pltpu.make_async_copy(table_hbm.at[idx_v.at[l]], rows[b],
                                  sgs[b]).wait()

        def out_slices(l, b):
            c = base + l
            h = c // n_bhi
            bhi = c % n_bhi
            return [(cells[b].at[q], out_hbm.at[h, q, bhi]) for q in range(fhi)]

        def fire_out(l, b):
            for src, dst in out_slices(l, b):
                pltpu.async_copy(src, dst, sos[b])

        def wait_out(l, b):
            for src, dst in out_slices(l, b):
                pltpu.make_async_copy(src, dst, sos[b]).wait()

        def transpose(b):
            rbuf = rows[b]
            cbuf = cells[b]
            for f in range(D):
                f_idx = jnp.full((_LANES,), f, jnp.int32)
                for bc in range(_CH // _LANES):
                    vals = plsc.load_gather(rbuf, [b_idx[bc], f_idx])
                    cbuf[f // 8, f % 8, pl.ds(bc * _LANES, _LANES)] = vals

        fire_gather(0, 0)
        fire_gather(1, 1)
        wait_gather(0, 0)
        transpose(0)
        fire_out(0, 0)

        def body(i, carry):
            l1 = 2 * i + 1
            wait_gather(l1, 1)
            fire_gather(l1 + 1, 0)
            transpose(1)
            wait_out(l1 - 1, 0)
            fire_out(l1, 1)
            l2 = 2 * i + 2
            wait_gather(l2, 0)
            fire_gather(l2 + 1, 1)
            transpose(0)
            wait_out(l2 - 1, 1)
            fire_out(l2, 0)
            return carry

        lax.fori_loop(0, (per_w - 2) // 2, body, 0)

        l_last = per_w - 1
        wait_gather(l_last, 1)
        transpose(1)
        wait_out(l_last - 1, 0)
        fire_out(l_last, 1)
        wait_out(l_last, 1)

    out5 = run(ids2, table)
    return out5.transpose(2, 4, 0, 1, 3).reshape(B, H, D)


# speedup vs baseline: 1.8917x; 1.8917x over previous
import functools

import jax
import jax.numpy as jnp
from jax import lax
from jax.experimental import pallas as pl
from jax.experimental.pallas import tpu as pltpu
from jax.experimental.pallas import tpu_sc as plsc

_NC = 2
_NS = 16
_NW = _NC * _NS

_CH = 128
_LANES = 16


def kernel(input_ids, table):
    B, H = input_ids.shape
    V, D = table.shape
    n_bhi = B // _CH
    n_cells = H * n_bhi
    per_w = n_cells // _NW
    fhi = D // 8
    blk_f = D // _LANES
    blk_b = _CH // _LANES
    cell_elems = _CH * D

    ids2 = input_ids.T.reshape(n_cells, _CH)

    mesh = plsc.VectorSubcoreMesh(core_axis_name="c", subcore_axis_name="s")

    @functools.partial(
        pl.kernel,
        out_type=jax.ShapeDtypeStruct((H, fhi, n_bhi, 8 * _CH), table.dtype),
        mesh=mesh,
        scratch_types=[
            pltpu.VMEM((per_w, _CH), jnp.int32),
            pltpu.VMEM((_CH, D), jnp.float32),
            pltpu.VMEM((_CH, D), jnp.float32),
            pltpu.VMEM((cell_elems,), jnp.float32),
            pltpu.VMEM((cell_elems,), jnp.float32),
            pltpu.SemaphoreType.DMA,
            pltpu.SemaphoreType.DMA,
            pltpu.SemaphoreType.DMA,
            pltpu.SemaphoreType.DMA,
        ],
        compiler_params=pltpu.CompilerParams(
            use_tc_tiling_on_sc=False, needs_layout_passes=False
        ),
    )
    def run(ids_hbm, table_hbm, out_hbm, idx_v, rows0, rows1, cell0, cell1,
            sg0, sg1, so0, so1):
        wid = lax.axis_index("s") * _NC + lax.axis_index("c")
        base = wid * per_w
        pltpu.sync_copy(ids_hbm.at[pl.ds(base, per_w)], idx_v)

        rows = (rows0, rows1)
        cells = (cell0, cell1)
        sgs = (sg0, sg1)
        sos = (so0, so1)

        iota = lax.iota(jnp.int32, _LANES)
        rot = [jnp.bitwise_and(iota + j, _LANES - 1) for j in range(_LANES)]
        sbase = [rot[j] * _CH + iota for j in range(_LANES)]

        def fire_gather(l, b):
            pltpu.async_copy(table_hbm.at[idx_v.at[l]], rows[b], sgs[b])

        def wait_gather(l, b):
            pltpu.make_async_copy(table_hbm.at[idx_v.at[l]], rows[b],
                                  sgs[b]).wait()

        def out_slices(l, b):
            c = base + l
            h = c // n_bhi
            bhi = c % n_bhi
            return [
                (cells[b].at[pl.ds(q * 8 * _CH, 8 * _CH)],
                 out_hbm.at[h, q, bhi])
                for q in range(fhi)
            ]

        def fire_out(l, b):
            for src, dst in out_slices(l, b):
                pltpu.async_copy(src, dst, sos[b])

        def wait_out(l, b):
            for src, dst in out_slices(l, b):
                pltpu.make_async_copy(src, dst, sos[b]).wait()

        def transpose(b):
            rbuf = rows[b]
            cbuf = cells[b]

            def blk(k, carry):
                f0 = (k // blk_b) * _LANES
                b0 = (k % blk_b) * _LANES
                bvec = iota + b0
                off = f0 * _CH + b0
                for j in range(_LANES):
                    vals = plsc.load_gather(rbuf, [bvec, rot[j] + f0])
                    plsc.store_scatter(cbuf, [sbase[j] + off], vals)
                return carry

            lax.fori_loop(0, blk_f * blk_b, blk, 0)

        fire_gather(0, 0)
        fire_gather(1, 1)
        wait_gather(0, 0)
        transpose(0)
        fire_out(0, 0)

        def body(i, carry):
            l1 = 2 * i + 1
            wait_gather(l1, 1)
            fire_gather(l1 + 1, 0)
            transpose(1)
            wait_out(l1 - 1, 0)
            fire_out(l1, 1)
            l2 = 2 * i + 2
            wait_gather(l2, 0)
            fire_gather(l2 + 1, 1)
            transpose(0)
            wait_out(l2 - 1, 1)
            fire_out(l2, 0)
            return carry

        lax.fori_loop(0, (per_w - 2) // 2, body, 0)

        l_last = per_w - 1
        wait_gather(l_last, 1)
        transpose(1)
        wait_out(l_last - 1, 0)
        fire_out(l_last, 1)
        wait_out(l_last, 1)

    out5 = run(ids2, table)
    out5 = out5.reshape(H, fhi, n_bhi, 8, _CH)
    return out5.transpose(2, 4, 0, 1, 3).reshape(B, H, D)
